# Initial kernel scaffold; baseline (speedup 1.0000x reference)
import jax
import jax.numpy as jnp
from jax.experimental import pallas as pl


def kernel(x, edge_index, edge_weight, W0, b0, Wc, Wout, bout):
    raise NotImplementedError("write your pallas kernel here")


# speedup vs baseline: 3.7006x; 3.7006x over previous
import functools
import math

import jax
import jax.numpy as jnp
from jax import lax
from jax.experimental import pallas as pl
from jax.experimental.pallas import tpu as pltpu
from jax.experimental.pallas import tpu_sc as plsc

_N = 10000
_E = 320000
_NHID = 128
_NCLASS = 64
_NLAYERS = 4
_LAMDA = 0.5
_ALPHA = 0.1

_NC = 2
_NS = 16
_NW = _NC * _NS
_CHUNK = 80
_EPT = _E // _NW
_NCHUNKS = _EPT // _CHUNK
_NPAD = 10240
_RPT = _NPAD // _NS
_ZROWS = 128
_LANES = 16


def _spmm_body(h_hbm, src_hbm, dst_hbm, w_hbm, out_hbm,
               idx_v, dst_v, w_v, rows_v, zero_v, acc_sh, sem):
    c = lax.axis_index("c")
    s = lax.axis_index("s")
    wid = c * _NS + s

    def _zrow(i, carry):
        for j in range(_NHID // _LANES):
            zero_v[i, pl.ds(j * _LANES, _LANES)] = jnp.zeros((_LANES,), jnp.float32)
        return carry
    lax.fori_loop(0, _ZROWS, _zrow, 0)
    for q in range(_RPT // _ZROWS):
        pltpu.sync_copy(zero_v, acc_sh.at[pl.ds(s * _RPT + q * _ZROWS, _ZROWS)])
    plsc.subcore_barrier()

    ebase = wid * _EPT

    def _chunk(t, carry):
        off = ebase + t * _CHUNK
        pltpu.sync_copy(src_hbm.at[pl.ds(off, _CHUNK)], idx_v)
        pltpu.sync_copy(dst_hbm.at[pl.ds(off, _CHUNK)], dst_v)
        pltpu.sync_copy(w_hbm.at[pl.ds(off, _CHUNK)], w_v)
        pltpu.async_copy(h_hbm.at[idx_v], rows_v, sem).wait()

        def _group(g, carry2):
            w16 = w_v[pl.ds(g * _LANES, _LANES)]
            for k in range(_LANES):
                wk = lax.gather(
                    w16, jnp.full((_LANES, 1), k, jnp.int32),
                    lax.GatherDimensionNumbers(
                        offset_dims=(), collapsed_slice_dims=(0,),
                        start_index_map=(0,)),
                    slice_sizes=(1,),
                    mode=lax.GatherScatterMode.PROMISE_IN_BOUNDS)
                e = g * _LANES + k
                for j in range(_NHID // _LANES):
                    sl = pl.ds(j * _LANES, _LANES)
                    rows_v[e, sl] = rows_v[e, sl] * wk
            return carry2
        lax.fori_loop(0, _CHUNK // _LANES, _group, 0)

        pltpu.sync_copy(rows_v, acc_sh.at[dst_v], add=True)
        return carry
    lax.fori_loop(0, _NCHUNKS, _chunk, 0)

    plsc.subcore_barrier()
    pltpu.sync_copy(acc_sh.at[pl.ds(s * _RPT, _RPT)],
                    out_hbm.at[c, pl.ds(s * _RPT, _RPT)])


def _make_spmm():
    mesh = plsc.VectorSubcoreMesh(core_axis_name="c", subcore_axis_name="s")
    return pl.kernel(
        _spmm_body,
        out_type=jax.ShapeDtypeStruct((_NC, _NPAD, _NHID), jnp.float32),
        mesh=mesh,
        scratch_types=[
            pltpu.VMEM((_CHUNK,), jnp.int32),
            pltpu.VMEM((_CHUNK,), jnp.int32),
            pltpu.VMEM((_CHUNK,), jnp.float32),
            pltpu.VMEM((_CHUNK, _NHID), jnp.float32),
            pltpu.VMEM((_ZROWS, _NHID), jnp.float32),
            pltpu.VMEM_SHARED((_NPAD, _NHID), jnp.float32),
            pltpu.SemaphoreType.DMA,
        ],
    )


_BN = 1000


def _fc0_body(x_ref, w_ref, b_ref, o_ref):
    t = jnp.dot(x_ref[...], w_ref[...], preferred_element_type=jnp.float32)
    o_ref[...] = jnp.maximum(t + b_ref[...], 0.0)


def _dense_body(theta, p_ref, h0_ref, h_ref, w_ref, o_ref):
    sup = (1.0 - _ALPHA) * (p_ref[0] + p_ref[1]) + _ALPHA * h0_ref[...]
    t = jnp.dot(sup, w_ref[...], preferred_element_type=jnp.float32)
    o_ref[...] = jnp.maximum(theta * t + (1.0 - theta) * sup + h_ref[...], 0.0)


def _final_body(h_ref, w_ref, b_ref, o_ref):
    t = jnp.dot(h_ref[...], w_ref[...], preferred_element_type=jnp.float32)
    o_ref[...] = jax.nn.sigmoid(t + b_ref[...])


def _fc0(x, W0, b0):
    return pl.pallas_call(
        _fc0_body,
        grid=(_N // _BN,),
        in_specs=[
            pl.BlockSpec((_BN, _NHID), lambda i: (i, 0)),
            pl.BlockSpec((_NHID, _NHID), lambda i: (0, 0)),
            pl.BlockSpec((1, _NHID), lambda i: (0, 0)),
        ],
        out_specs=pl.BlockSpec((_BN, _NHID), lambda i: (i, 0)),
        out_shape=jax.ShapeDtypeStruct((_N, _NHID), jnp.float32),
    )(x, W0, b0.reshape(1, _NHID))


def _dense(p, h0, h, W, theta):
    return pl.pallas_call(
        functools.partial(_dense_body, theta),
        grid=(_N // _BN,),
        in_specs=[
            pl.BlockSpec((_NC, _BN, _NHID), lambda i: (0, i, 0)),
            pl.BlockSpec((_BN, _NHID), lambda i: (i, 0)),
            pl.BlockSpec((_BN, _NHID), lambda i: (i, 0)),
            pl.BlockSpec((_NHID, _NHID), lambda i: (0, 0)),
        ],
        out_specs=pl.BlockSpec((_BN, _NHID), lambda i: (i, 0)),
        out_shape=jax.ShapeDtypeStruct((_N, _NHID), jnp.float32),
    )(p, h0, h, W)


def _final(h, Wout, bout):
    return pl.pallas_call(
        _final_body,
        grid=(_N // _BN,),
        in_specs=[
            pl.BlockSpec((_BN, _NHID), lambda i: (i, 0)),
            pl.BlockSpec((_NHID, _NCLASS), lambda i: (0, 0)),
            pl.BlockSpec((1, _NCLASS), lambda i: (0, 0)),
        ],
        out_specs=pl.BlockSpec((_BN, _NCLASS), lambda i: (i, 0)),
        out_shape=jax.ShapeDtypeStruct((_N, _NCLASS), jnp.float32),
    )(h, Wout, bout.reshape(1, _NCLASS))


def kernel(x, edge_index, edge_weight, W0, b0, Wc, Wout, bout):
    src = edge_index[0]
    dst = edge_index[1]
    spmm = _make_spmm()
    h = _fc0(x, W0, b0)
    h0 = h
    for i in range(_NLAYERS):
        theta = math.log(_LAMDA / (i + 1) + 1.0)
        p = spmm(h, src, dst, edge_weight)
        h = _dense(p, h0, h, Wc[i], theta)
    return _final(h, Wout, bout)
